# Initial kernel scaffold; baseline (speedup 1.0000x reference)
import jax
import jax.numpy as jnp
from jax.experimental import pallas as pl


def kernel(x, vec, edge_index, edge_rbf, edge_vector, W1, b1, W2, b2, We, be):
    raise NotImplementedError("write your pallas kernel here")


# speedup vs baseline: 9.0230x; 9.0230x over previous
import functools
import math

import jax
import jax.numpy as jnp
import numpy as np
from jax import lax
from jax.experimental import pallas as pl
from jax.experimental.pallas import tpu as pltpu
from jax.experimental.pallas import tpu_sc as plsc

H = 128
G = 4
GF = 32
GC = 3 * GF
TROW = 2 * GC
MROW = 4 * GF
NC, NS = 2, 16
CHUNK = 80


def _node_mlp_body(x_ref, w1_ref, b1_ref, w2_ref, b2_ref, o_ref):
    xb = x_ref[...]
    h = jnp.dot(xb, w1_ref[...].T, preferred_element_type=jnp.float32)
    h = h + b1_ref[...]
    h = (h * jax.nn.sigmoid(h)) * (1.0 / 0.6)
    y = jnp.dot(h, w2_ref[...].T, preferred_element_type=jnp.float32)
    o_ref[...] = y + b2_ref[...]


def _edge_proj_body(r_ref, we_ref, be_ref, o_ref):
    r = r_ref[...]
    for g in range(G):
        y = jnp.dot(r, we_ref[pl.ds(g * GC, GC), :].T,
                    preferred_element_type=jnp.float32)
        o_ref[g, :, :] = y + be_ref[pl.ds(g, 1), :]


def _node_mlp(x, W1, b1, W2P, b2P):
    n = x.shape[0]
    bn = 2000
    return pl.pallas_call(
        _node_mlp_body,
        grid=(n // bn,),
        in_specs=[
            pl.BlockSpec((bn, H), lambda m: (m, 0)),
            pl.BlockSpec(W1.shape, lambda m: (0, 0)),
            pl.BlockSpec((1, H // 2), lambda m: (0, 0)),
            pl.BlockSpec(W2P.shape, lambda m: (0, 0)),
            pl.BlockSpec((1, 3 * H), lambda m: (0, 0)),
        ],
        out_specs=pl.BlockSpec((bn, 3 * H), lambda m: (m, 0)),
        out_shape=jax.ShapeDtypeStruct((n, 3 * H), jnp.float32),
    )(x, W1, b1.reshape(1, -1), W2P, b2P.reshape(1, -1))


def _edge_proj(edge_rbf, WeP, beP):
    e = edge_rbf.shape[0]
    be_blk = 4000
    return pl.pallas_call(
        _edge_proj_body,
        grid=(e // be_blk,),
        in_specs=[
            pl.BlockSpec((be_blk, edge_rbf.shape[1]), lambda m: (m, 0)),
            pl.BlockSpec(WeP.shape, lambda m: (0, 0)),
            pl.BlockSpec((G, GC), lambda m: (0, 0)),
        ],
        out_specs=pl.BlockSpec((G, be_blk, GC), lambda m: (0, m, 0)),
        out_shape=jax.ShapeDtypeStruct((G, e, GC), jnp.float32),
    )(edge_rbf, WeP, beP.reshape(G, GC))


def _sc_body(n_pad, n_edges,
             t2_hbm, rbf_hbm, ev_hbm, ej_hbm, ei_hbm, out_hbm,
             acc, idxj_v, idxt_v, idxi_v, t_v, rbf_v, ev_v, msg_v, zrow_v,
             sem):
    c = lax.axis_index("c")
    s = lax.axis_index("s")
    e_per_tile = n_edges // NS
    n_chunks = e_per_tile // CHUNK
    rows_per_tile = n_pad // NS
    zrows = zrow_v.shape[0]
    n_zcopies = rows_per_tile // zrows

    def _zfill16(k, _):
        r = k // (MROW // 16)
        col = (k % (MROW // 16)) * 16
        zrow_v[r, pl.ds(col, 16)] = jnp.zeros((16,), jnp.float32)
        return 0
    lax.fori_loop(0, zrows * (MROW // 16), _zfill16, 0)

    row0 = s * rows_per_tile

    for p in range(2):
        g = c * 2 + p

        def _zero(k, _):
            pltpu.sync_copy(zrow_v, acc.at[pl.ds(row0 + k * zrows, zrows), :])
            return 0
        lax.fori_loop(0, n_zcopies, _zero, 0)
        plsc.subcore_barrier()

        def _chunk(ci, _):
            e0 = pl.multiple_of(s * e_per_tile + ci * CHUNK, CHUNK)
            pltpu.sync_copy(ej_hbm.at[pl.ds(e0, CHUNK)], idxj_v)
            pltpu.sync_copy(ei_hbm.at[pl.ds(e0, CHUNK)], idxi_v)
            def _mkidx(k, _):
                idxt_v[pl.ds(k * 16, 16)] = idxj_v[pl.ds(k * 16, 16)] * G + g
                return 0
            lax.fori_loop(0, CHUNK // 16, _mkidx, 0)
            gather = pltpu.async_copy(t2_hbm.at[idxt_v], t_v, sem)
            pltpu.sync_copy(rbf_hbm.at[g, pl.ds(e0, CHUNK), :], rbf_v)
            pltpu.sync_copy(ev_hbm.at[pl.ds(e0 * 3, CHUNK * 3)],
                            ev_v.at[pl.ds(0, CHUNK * 3)])
            gather.wait()

            def _edge(e, _):
                evw = ev_v[pl.ds(e * 3, 16)]
                ev0 = evw[0]
                ev1 = evw[1]
                ev2 = evw[2]
                for k in range(GF // 16):
                    xh1 = t_v[e, pl.ds(k * 16, 16)]
                    xh2 = t_v[e, pl.ds(32 + k * 16, 16)]
                    xh3 = t_v[e, pl.ds(64 + k * 16, 16)]
                    v0 = t_v[e, pl.ds(96 + k * 16, 16)]
                    v1 = t_v[e, pl.ds(128 + k * 16, 16)]
                    v2 = t_v[e, pl.ds(160 + k * 16, 16)]
                    rb1 = rbf_v[e, pl.ds(k * 16, 16)]
                    rb2 = rbf_v[e, pl.ds(32 + k * 16, 16)]
                    rb3 = rbf_v[e, pl.ds(64 + k * 16, 16)]
                    t1 = xh1 * rb1
                    t2 = xh2 * rb2
                    msg_v[e, pl.ds(k * 16, 16)] = xh3 * rb3
                    msg_v[e, pl.ds(32 + k * 16, 16)] = t1 * v0 + t2 * ev0
                    msg_v[e, pl.ds(64 + k * 16, 16)] = t1 * v1 + t2 * ev1
                    msg_v[e, pl.ds(96 + k * 16, 16)] = t1 * v2 + t2 * ev2
                return 0
            lax.fori_loop(0, CHUNK, _edge, 0)

            pltpu.sync_copy(msg_v, acc.at[idxi_v], add=True)
            return 0
        lax.fori_loop(0, n_chunks, _chunk, 0)
        plsc.subcore_barrier()

        pltpu.sync_copy(acc.at[pl.ds(row0, rows_per_tile), :],
                        out_hbm.at[g, pl.ds(row0, rows_per_tile), :])
        plsc.subcore_barrier()


def _sc_call(t2, rbf_hp, ev, ej, ei, n_pad, n_edges):
    mesh = plsc.VectorSubcoreMesh(core_axis_name="c", subcore_axis_name="s")
    body = functools.partial(_sc_body, n_pad, n_edges)
    return pl.kernel(
        body,
        out_type=jax.ShapeDtypeStruct((G, n_pad, MROW), jnp.float32),
        mesh=mesh,
        compiler_params=pltpu.CompilerParams(use_tc_tiling_on_sc=False),
        scratch_types=[
            pltpu.VMEM_SHARED((n_pad, MROW), jnp.float32),
            pltpu.VMEM((CHUNK,), jnp.int32),
            pltpu.VMEM((CHUNK,), jnp.int32),
            pltpu.VMEM((CHUNK,), jnp.int32),
            pltpu.VMEM((CHUNK, TROW), jnp.float32),
            pltpu.VMEM((CHUNK, GC), jnp.float32),
            pltpu.VMEM((CHUNK * 3 + 16,), jnp.float32),
            pltpu.VMEM((CHUNK, MROW), jnp.float32),
            pltpu.VMEM((64, MROW), jnp.float32),
            pltpu.SemaphoreType.DMA,
        ],
    )(t2, rbf_hp, ev, ej, ei)


def _perm_and_scales():
    p = np.zeros(3 * H, dtype=np.int32)
    for g in range(G):
        for t in range(GC):
            p[GC * g + t] = (t // GF) * H + GF * g + (t % GF)
    inv3 = 1.0 / math.sqrt(3.0)
    invh = 1.0 / math.sqrt(H)
    s = np.where(np.arange(GC) < 2 * GF, inv3 * invh, inv3).astype(np.float32)
    return p, np.tile(s, G)


_P, _SFULL = _perm_and_scales()


def kernel(x, vec, edge_index, edge_rbf, edge_vector, W1, b1, W2, b2, We, be):
    n = x.shape[0]
    e = edge_rbf.shape[0]
    p = jnp.asarray(_P)
    sf = jnp.asarray(_SFULL)

    W2P, b2P = W2[p], b2[p]
    WeP = We[p] * sf[:, None]
    beP = be[p] * sf

    x_hp = _node_mlp(x, W1, b1, W2P, b2P)
    rbf_hp = _edge_proj(edge_rbf, WeP, beP)

    vecP = vec.reshape(n, 3, G, GF).transpose(0, 2, 1, 3).reshape(n, G, GC)
    t2 = jnp.concatenate([x_hp.reshape(n, G, GC), vecP], axis=2)
    t2 = t2.reshape(n * G, TROW)

    ej = edge_index[0].astype(jnp.int32)
    ei = edge_index[1].astype(jnp.int32)

    n_pad = ((n + NS * 128 - 1) // (NS * 128)) * (NS * 128)
    outg = _sc_call(t2, rbf_hp, edge_vector.reshape(-1), ej, ei, n_pad, e)
    outg = outg[:, :n]

    d_x = outg[:, :, 0:GF].transpose(1, 0, 2).reshape(n, H)
    d_vec = (outg[:, :, GF:MROW].reshape(G, n, 3, GF)
             .transpose(1, 2, 0, 3).reshape(n, 3, H))
    return (d_x, d_vec)
